# baseline (speedup 1.0000x reference)
import numpy as np

import jax
import jax.numpy as jnp
from jax.experimental import pallas as pl
from jax.experimental.pallas import tpu as pltpu

_AFWD = 0.999
_EPS = 1e-05
_R = 256
_RB = 512
_LB = 4096


def _tri_consts():
    idx = np.arange(_R)
    dd = idx[:, None] - idx[None, :]
    t = np.where(dd >= 0, _AFWD ** np.maximum(dd, 0), 0.0).astype(np.float32)
    t2 = np.where(dd >= 1, _AFWD ** np.maximum(dd - 1, 0), 0.0).astype(np.float32)
    dcol = (_AFWD ** idx).astype(np.float32).reshape(_R, 1)
    return t, t2, dcol


def _body(t_hi_ref, t_lo_ref, t2_ref, dcol_ref, m_ref, v_ref, x_ref, o_ref,
          mu_s, var_s):
    ci = pl.program_id(1)

    @pl.when(ci == 0)
    def _():
        mu_s[...] = m_ref[...]
        var_s[...] = v_ref[...]

    a = _AFWD
    c = a * (1.0 - a)
    t_hi = t_hi_ref[...]
    t_lo = t_lo_ref[...]
    t2 = t2_ref[...]
    dcol = dcol_ref[...]
    mu0 = mu_s[...]
    v0 = var_s[...]

    x = x_ref[...]
    for s in range(_RB // _R):
        xs = x[s * _R:(s + 1) * _R]
        y = xs - jnp.concatenate([mu0, xs[:-1]], axis=0)
        y_h = y.astype(jnp.bfloat16)
        y_l = (y - y_h.astype(jnp.float32)).astype(jnp.bfloat16)
        d = (jnp.dot(t_hi, y_h, preferred_element_type=jnp.float32)
             + jnp.dot(t_lo, y_h, preferred_element_type=jnp.float32)
             + jnp.dot(t_hi, y_l, preferred_element_type=jnp.float32))
        w = c * (d * d)
        u = jnp.dot(t2, w.astype(jnp.bfloat16),
                    preferred_element_type=jnp.float32)
        vprev = u + dcol * v0
        o_ref[s * _R:(s + 1) * _R, :] = d * jax.lax.rsqrt(vprev + _EPS)
        mu0 = xs[-1:] - a * d[-1:]
        v0 = a * vprev[-1:] + w[-1:]

    mu_s[...] = mu0
    var_s[...] = v0


def kernel(x, m, var):
    n, l = x.shape
    t, t2, dcol = _tri_consts()
    t_f32 = jnp.asarray(t)
    t_hi = t_f32.astype(jnp.bfloat16)
    t_lo = (t_f32 - t_hi.astype(jnp.float32)).astype(jnp.bfloat16)
    t2_b = jnp.asarray(t2).astype(jnp.bfloat16)
    dcol_j = jnp.asarray(dcol)
    m2 = m.reshape(1, l)
    v2 = var.reshape(1, l)
    grid = (l // _LB, n // _RB)
    out = pl.pallas_call(
        _body,
        out_shape=jax.ShapeDtypeStruct((n, l), x.dtype),
        grid=grid,
        in_specs=[
            pl.BlockSpec((_R, _R), lambda li, ci: (0, 0)),
            pl.BlockSpec((_R, _R), lambda li, ci: (0, 0)),
            pl.BlockSpec((_R, _R), lambda li, ci: (0, 0)),
            pl.BlockSpec((_R, 1), lambda li, ci: (0, 0)),
            pl.BlockSpec((1, _LB), lambda li, ci: (0, li)),
            pl.BlockSpec((1, _LB), lambda li, ci: (0, li)),
            pl.BlockSpec((_RB, _LB), lambda li, ci: (ci, li)),
        ],
        out_specs=pl.BlockSpec((_RB, _LB), lambda li, ci: (ci, li)),
        scratch_shapes=[
            pltpu.VMEM((1, _LB), jnp.float32),
            pltpu.VMEM((1, _LB), jnp.float32),
        ],
        compiler_params=pltpu.CompilerParams(
            dimension_semantics=("parallel", "arbitrary"),
        ),
        name="control_norm1d_mxu",
    )(t_hi, t_lo, t2_b, dcol_j, m2, v2, x)
    return out


# speedup vs baseline: 1.0710x; 1.0710x over previous
import numpy as np

import jax
import jax.numpy as jnp
from jax.experimental import pallas as pl
from jax.experimental.pallas import tpu as pltpu

_AFWD = 0.999
_EPS = 1e-05
_R = 256
_RB = 2048
_LB = 1024


def _tri_consts():
    idx = np.arange(_R)
    dd = idx[:, None] - idx[None, :]
    t = np.where(dd >= 0, _AFWD ** np.maximum(dd, 0), 0.0).astype(np.float32)
    t2 = np.where(dd >= 1, _AFWD ** np.maximum(dd - 1, 0), 0.0).astype(np.float32)
    dcol = (_AFWD ** idx).astype(np.float32).reshape(_R, 1)
    return t, t2, dcol


def _body(t_hi_ref, t_lo_ref, t2_ref, dcol_ref, m_ref, v_ref, x_ref, o_ref,
          mu_s, var_s):
    ci = pl.program_id(1)

    @pl.when(ci == 0)
    def _():
        mu_s[...] = m_ref[...]
        var_s[...] = v_ref[...]

    a = _AFWD
    c = a * (1.0 - a)
    t_hi = t_hi_ref[...]
    t_lo = t_lo_ref[...]
    t2 = t2_ref[...]
    dcol = dcol_ref[...]
    mu0 = mu_s[...]
    v0 = var_s[...]

    x = x_ref[...]
    for s in range(_RB // _R):
        xs = x[s * _R:(s + 1) * _R]
        y = xs - jnp.concatenate([mu0, xs[:-1]], axis=0)
        y_h = y.astype(jnp.bfloat16)
        y_l = (y - y_h.astype(jnp.float32)).astype(jnp.bfloat16)
        d = (jnp.dot(t_hi, y_h, preferred_element_type=jnp.float32)
             + jnp.dot(t_lo, y_h, preferred_element_type=jnp.float32)
             + jnp.dot(t_hi, y_l, preferred_element_type=jnp.float32))
        w = c * (d * d)
        u = jnp.dot(t2, w.astype(jnp.bfloat16),
                    preferred_element_type=jnp.float32)
        vprev = u + dcol * v0
        o_ref[s * _R:(s + 1) * _R, :] = d * jax.lax.rsqrt(vprev + _EPS)
        mu0 = xs[-1:] - a * d[-1:]
        v0 = a * vprev[-1:] + w[-1:]

    mu_s[...] = mu0
    var_s[...] = v0


def kernel(x, m, var):
    n, l = x.shape
    t, t2, dcol = _tri_consts()
    t_f32 = jnp.asarray(t)
    t_hi = t_f32.astype(jnp.bfloat16)
    t_lo = (t_f32 - t_hi.astype(jnp.float32)).astype(jnp.bfloat16)
    t2_b = jnp.asarray(t2).astype(jnp.bfloat16)
    dcol_j = jnp.asarray(dcol)
    m2 = m.reshape(1, l)
    v2 = var.reshape(1, l)
    grid = (l // _LB, n // _RB)
    out = pl.pallas_call(
        _body,
        out_shape=jax.ShapeDtypeStruct((n, l), x.dtype),
        grid=grid,
        in_specs=[
            pl.BlockSpec((_R, _R), lambda li, ci: (0, 0)),
            pl.BlockSpec((_R, _R), lambda li, ci: (0, 0)),
            pl.BlockSpec((_R, _R), lambda li, ci: (0, 0)),
            pl.BlockSpec((_R, 1), lambda li, ci: (0, 0)),
            pl.BlockSpec((1, _LB), lambda li, ci: (0, li)),
            pl.BlockSpec((1, _LB), lambda li, ci: (0, li)),
            pl.BlockSpec((_RB, _LB), lambda li, ci: (ci, li)),
        ],
        out_specs=pl.BlockSpec((_RB, _LB), lambda li, ci: (ci, li)),
        scratch_shapes=[
            pltpu.VMEM((1, _LB), jnp.float32),
            pltpu.VMEM((1, _LB), jnp.float32),
        ],
        compiler_params=pltpu.CompilerParams(
            dimension_semantics=("parallel", "arbitrary"),
        ),
        name="control_norm1d_mxu",
    )(t_hi, t_lo, t2_b, dcol_j, m2, v2, x)
    return out
